# Initial kernel scaffold; baseline (speedup 1.0000x reference)
import jax
import jax.numpy as jnp
from jax.experimental import pallas as pl


def kernel(x, table):
    raise NotImplementedError("write your pallas kernel here")


# speedup vs baseline: 1.8000x; 1.8000x over previous
import jax
import jax.numpy as jnp
from jax.experimental import pallas as pl

MAX_LEN = 8192
EMBED_DIM = 768
BATCH = 4
SBLK = 1024


def _add_body(x_ref, t_ref, o_ref):
    o_ref[...] = x_ref[...] + t_ref[...][None, :, :]


def kernel(x, table):
    grid = (MAX_LEN // SBLK,)
    return pl.pallas_call(
        _add_body,
        grid=grid,
        in_specs=[
            pl.BlockSpec((BATCH, SBLK, EMBED_DIM), lambda i: (0, i, 0)),
            pl.BlockSpec((SBLK, EMBED_DIM), lambda i: (i, 0)),
        ],
        out_specs=pl.BlockSpec((BATCH, SBLK, EMBED_DIM), lambda i: (0, i, 0)),
        out_shape=jax.ShapeDtypeStruct((BATCH, MAX_LEN, EMBED_DIM), jnp.float32),
    )(x, table)
